# baseline (speedup 1.0000x reference)
import functools

import jax
import jax.numpy as jnp
from jax import lax
from jax.experimental import pallas as pl
from jax.experimental.pallas import tpu as pltpu
from jax.experimental.pallas import tpu_sc as plsc

EMBED_DIM = 32
NC = 2
NS = 16
NW = NC * NS
ST = 25
BT = 128
TILES = ST * BT
TPW = TILES // NW
NU = 2 * TPW
ROWS_U = 512


def _body(rel5_hbm, table_hbm, out5_hbm, idx_v, rows_v, trans_v,
          sem_i, sem_g, sem_o):
    wid = lax.axis_index("s") * NC + lax.axis_index("c")
    tile0 = wid * TPW

    iota16 = lax.iota(jnp.int32, 16)

    def unit_addr(k):
        tile = tile0 + k // 2
        st = tile // BT
        bt = tile % BT
        s0 = st * 8 + (k % 2) * 4
        return st, bt, s0

    def idx_slice(k):
        st, bt, _ = unit_addr(k)
        return rel5_hbm.at[st, bt, pl.ds((k % 2) * 4, 4), :]

    def fire_idx(k, b):
        pltpu.async_copy(idx_slice(k), idx_v.at[b], sem_i.at[b])

    def wait_idx(k, b):
        pltpu.make_async_copy(idx_slice(k), idx_v.at[b], sem_i.at[b]).wait()

    def fire_gathers(b):
        for si in range(4):
            pltpu.async_copy(
                table_hbm.at[idx_v.at[b, si]],
                rows_v.at[b, pl.ds(si * 128, 128), :],
                sem_g.at[b],
            )

    def drain_gathers(b):
        pltpu.make_async_copy(
            table_hbm.at[pl.ds(0, ROWS_U), :], rows_v.at[b], sem_g.at[b]
        ).wait()

    def fire_writes(k, b):
        _, bt, s0 = unit_addr(k)
        for et in range(4):
            pltpu.async_copy(
                trans_v.at[b, et],
                out5_hbm.at[pl.ds(s0, 4), et, bt, :, :],
                sem_o.at[b],
            )

    def drain_writes(b):
        pltpu.make_async_copy(
            out5_hbm.at[pl.ds(0, 4), :, 0, :, :], trans_v.at[b], sem_o.at[b]
        ).wait()

    def transpose(b):
        def tbody(i):
            sig = i // 32
            e = i % 32
            si = sig // 8
            g = sig % 8
            et = e // 8
            ei = e % 8
            row = iota16 + sig * 16
            col = jnp.full((16,), 0, jnp.int32) + e
            val = plsc.load_gather(rows_v.at[b], [row, col])
            trans_v[b, et, si, ei, pl.ds(g * 16, 16)] = val

        plsc.parallel_loop(0, 1024, 1, unroll=16)(tbody)

    pltpu.sync_copy(idx_slice(0), idx_v.at[0])
    fire_gathers(0)
    fire_idx(1, 1)

    def step(j, carry):
        k0 = 2 * j
        drain_gathers(0)

        @pl.when(j >= 1)
        def _():
            drain_writes(0)

        wait_idx(k0 + 1, 1)
        fire_gathers(1)

        @pl.when(k0 + 2 < NU)
        def _():
            fire_idx(k0 + 2, 0)

        transpose(0)
        fire_writes(k0, 0)

        k1 = k0 + 1
        drain_gathers(1)

        @pl.when(j >= 1)
        def _():
            drain_writes(1)

        @pl.when(k1 + 1 < NU)
        def _():
            wait_idx(k1 + 1, 0)
            fire_gathers(0)

        @pl.when(k1 + 2 < NU)
        def _():
            fire_idx(k1 + 2, 1)

        transpose(1)
        fire_writes(k1, 1)
        return carry

    lax.fori_loop(0, NU // 2, step, 0)

    drain_writes(0)
    drain_writes(1)


@jax.jit
def _gather5(rel5, table):
    mesh = plsc.VectorSubcoreMesh(core_axis_name="c", subcore_axis_name="s")
    k = pl.kernel(
        _body,
        out_type=jax.ShapeDtypeStruct((200, 4, 128, 8, 128), jnp.float32),
        mesh=mesh,
        scratch_types=[
            pltpu.VMEM((2, 4, 128), jnp.int32),
            pltpu.VMEM((2, ROWS_U, EMBED_DIM), jnp.float32),
            pltpu.VMEM((2, 4, 4, 8, 128), jnp.float32),
            pltpu.SemaphoreType.DMA((2,)),
            pltpu.SemaphoreType.DMA((2,)),
            pltpu.SemaphoreType.DMA((2,)),
        ],
        compiler_params=pltpu.CompilerParams(
            use_tc_tiling_on_sc=False, needs_layout_passes=False
        ),
    )
    return k(rel5, table)


def kernel(relation, table):
    rel5 = relation.reshape(128, 128, 25, 8).transpose(2, 0, 3, 1)
    out5 = _gather5(rel5.astype(jnp.int32), table)
    return out5.transpose(2, 4, 0, 1, 3).reshape(16384, 200, 32)


# speedup vs baseline: 1.0396x; 1.0396x over previous
import jax
import jax.numpy as jnp
from jax import lax
from jax.experimental import pallas as pl
from jax.experimental.pallas import tpu as pltpu
from jax.experimental.pallas import tpu_sc as plsc

EMBED_DIM = 32
NC = 2
NS = 16
NW = NC * NS
ST = 25
BT = 128
TILES = ST * BT
TPW = TILES // NW
NU = 2 * TPW
ROWS_U = 512


def _body(rel5_hbm, table_hbm, out5_hbm, idx_v, rows_v, trans_v,
          sem_i, sem_g, sem_o):
    wid = lax.axis_index("s") * NC + lax.axis_index("c")
    tile0 = wid * TPW

    iota16 = lax.iota(jnp.int32, 16)

    def unit_addr(k):
        tile = tile0 + k // 2
        st = tile // BT
        bt = tile % BT
        s0 = st * 8 + (k % 2) * 4
        return st, bt, s0

    def idx_slice(k):
        st, bt, _ = unit_addr(k)
        return rel5_hbm.at[st, bt, pl.ds((k % 2) * 4, 4), :]

    def fire_idx(k, b):
        pltpu.async_copy(idx_slice(k), idx_v.at[b], sem_i.at[b])

    def wait_idx(k, b):
        pltpu.make_async_copy(idx_slice(k), idx_v.at[b], sem_i.at[b]).wait()

    def fire_gathers(b):
        for si in range(4):
            pltpu.async_copy(
                table_hbm.at[idx_v.at[b, si]],
                rows_v.at[b, pl.ds(si * 128, 128), :],
                sem_g.at[b],
            )

    def drain_gathers(b):
        pltpu.make_async_copy(
            table_hbm.at[pl.ds(0, ROWS_U), :], rows_v.at[b], sem_g.at[b]
        ).wait()

    def fire_writes(k, b):
        _, bt, s0 = unit_addr(k)
        for et in range(4):
            pltpu.async_copy(
                trans_v.at[b, et],
                out5_hbm.at[pl.ds(s0, 4), et, bt, :, :],
                sem_o.at[b],
            )

    def drain_writes(b):
        pltpu.make_async_copy(
            out5_hbm.at[pl.ds(0, 4), :, 0, :, :], trans_v.at[b], sem_o.at[b]
        ).wait()

    def transpose(b):
        def tbody(i):
            sig = i // 32
            e = i % 32
            si = sig // 8
            g = sig % 8
            et = e // 8
            ei = e % 8
            row = iota16 + sig * 16
            col = jnp.full((16,), 0, jnp.int32) + e
            val = plsc.load_gather(rows_v.at[b], [row, col])
            trans_v[b, et, si, ei, pl.ds(g * 16, 16)] = val

        plsc.parallel_loop(0, 1024, 1, unroll=8)(tbody)

    pltpu.sync_copy(idx_slice(0), idx_v.at[0])
    fire_gathers(0)
    fire_idx(1, 1)

    def step(j, carry):
        k0 = 2 * j
        drain_gathers(0)

        @pl.when(j >= 1)
        def _():
            drain_writes(0)

        wait_idx(k0 + 1, 1)
        fire_gathers(1)

        @pl.when(k0 + 2 < NU)
        def _():
            fire_idx(k0 + 2, 0)

        transpose(0)
        fire_writes(k0, 0)

        k1 = k0 + 1
        drain_gathers(1)

        @pl.when(j >= 1)
        def _():
            drain_writes(1)

        @pl.when(k1 + 1 < NU)
        def _():
            wait_idx(k1 + 1, 0)
            fire_gathers(0)

        @pl.when(k1 + 2 < NU)
        def _():
            fire_idx(k1 + 2, 1)

        transpose(1)
        fire_writes(k1, 1)
        return carry

    lax.fori_loop(0, NU // 2, step, 0)

    drain_writes(0)
    drain_writes(1)


@jax.jit
def _gather5(rel5, table):
    mesh = plsc.VectorSubcoreMesh(core_axis_name="c", subcore_axis_name="s")
    k = pl.kernel(
        _body,
        out_type=jax.ShapeDtypeStruct((200, 4, 128, 8, 128), jnp.float32),
        mesh=mesh,
        scratch_types=[
            pltpu.VMEM((2, 4, 128), jnp.int32),
            pltpu.VMEM((2, ROWS_U, EMBED_DIM), jnp.float32),
            pltpu.VMEM((2, 4, 4, 8, 128), jnp.float32),
            pltpu.SemaphoreType.DMA((2,)),
            pltpu.SemaphoreType.DMA((2,)),
            pltpu.SemaphoreType.DMA((2,)),
        ],
        compiler_params=pltpu.CompilerParams(
            use_tc_tiling_on_sc=False, needs_layout_passes=False
        ),
    )
    return k(rel5, table)


def kernel(relation, table):
    rel5 = relation.reshape(128, 128, 25, 8).transpose(2, 0, 3, 1)
    out5 = _gather5(rel5.astype(jnp.int32), table)
    return out5.transpose(2, 4, 0, 1, 3).reshape(16384, 200, 32)
